# baseline (speedup 1.0000x reference)
import functools

import jax
import jax.numpy as jnp
from jax import lax
from jax.experimental import pallas as pl
from jax.experimental.pallas import tpu as pltpu
from jax.experimental.pallas import tpu_sc as plsc

BATCH = 16384
_INFO = plsc.get_sparse_core_info()
_NC, _NS, _L = _INFO.num_cores, _INFO.num_subcores, _INFO.num_lanes
_NW = _NC * _NS
_BPW = BATCH // _NW
_QTR = _BPW // 4


def _irt_body(stu_hbm, exer_hbm, theta_hbm, a_hbm, b_hbm, out_hbm,
              stu_v, exer_v, th_v, a_v, b_v, out_v,
              q0, q1, q2, q3, sem_o):
    sem_q = (q0, q1, q2, q3)
    wid = lax.axis_index("s") * _NC + lax.axis_index("c")
    base = wid * _BPW
    idx_copies = []
    for q, sem in enumerate(sem_q):
        qsl = pl.ds(q * _QTR, _QTR)
        hsl = pl.ds(base + q * _QTR, _QTR)
        idx_copies.append((
            pltpu.async_copy(stu_hbm.at[hsl], stu_v.at[0, qsl], sem),
            pltpu.async_copy(exer_hbm.at[hsl], exer_v.at[0, qsl], sem),
        ))
    gathers = []
    for q, sem in enumerate(sem_q):
        i1, i2 = idx_copies[q]
        i1.wait()
        i2.wait()
        qsl = pl.ds(q * _QTR, _QTR)
        gathers.append((
            pltpu.async_copy(a_hbm.at[exer_v.at[:, qsl]], a_v.at[:, qsl], sem),
            pltpu.async_copy(theta_hbm.at[stu_v.at[:, qsl]], th_v.at[:, qsl], sem),
            pltpu.async_copy(b_hbm.at[exer_v.at[:, qsl]], b_v.at[:, qsl], sem),
        ))
    outs = []
    for q, (ga, gth, gb) in enumerate(gathers):
        lo = q * _QTR
        ga.wait()
        for j in range(_QTR // _L):
            sl = pl.ds(lo + j * _L, _L)
            a_v[0, sl] = jnp.exp(-a_v[0, sl])
        gth.wait()
        gb.wait()
        for j in range(_QTR // _L):
            sl = pl.ds(lo + j * _L, _L)
            z = 3.4 * (th_v[0, sl] - b_v[0, sl]) / (1.0 + a_v[0, sl])
            th_v[0, sl] = jnp.exp(-z)
        for j in range(_QTR // _L):
            sl = pl.ds(lo + j * _L, _L)
            out_v[sl] = 1.0 / (1.0 + th_v[0, sl])
        outs.append(pltpu.async_copy(
            out_v.at[pl.ds(lo, _QTR)], out_hbm.at[pl.ds(base + lo, _QTR)], sem_o))
    for o in outs:
        o.wait()


_irt_sc = functools.partial(
    pl.kernel,
    mesh=plsc.VectorSubcoreMesh(core_axis_name="c", subcore_axis_name="s"),
    out_type=jax.ShapeDtypeStruct((BATCH,), jnp.float32),
    scratch_types=[
        pltpu.VMEM((1, _BPW), jnp.int32),
        pltpu.VMEM((1, _BPW), jnp.int32),
        pltpu.VMEM((1, _BPW), jnp.float32),
        pltpu.VMEM((1, _BPW), jnp.float32),
        pltpu.VMEM((1, _BPW), jnp.float32),
        pltpu.VMEM((_BPW,), jnp.float32),
        pltpu.SemaphoreType.DMA,
        pltpu.SemaphoreType.DMA,
        pltpu.SemaphoreType.DMA,
        pltpu.SemaphoreType.DMA,
        pltpu.SemaphoreType.DMA,
    ],
)(_irt_body)


def kernel(stu_id, exer_id, theta_w, a_w, b_w):
    return _irt_sc(
        stu_id.astype(jnp.int32),
        exer_id.astype(jnp.int32),
        theta_w.reshape(1, -1),
        a_w.reshape(1, -1),
        b_w.reshape(1, -1),
    )
